# baseline (speedup 1.0000x reference)
import functools

import jax
import jax.numpy as jnp
from jax import lax
from jax.experimental import pallas as pl
from jax.experimental.pallas import tpu as pltpu
from jax.experimental.pallas import tpu_sc as plsc

E = 160000
D = 256
HD = D // 2
N_DST = 10000
PAD_N = 10240


BM = 1280


def _mmT_body(w_ref, x_ref, b_ref, o_ref):
    acc = lax.dot_general(w_ref[...].astype(jnp.bfloat16),
                          x_ref[...].astype(jnp.bfloat16),
                          (((1,), (1,)), ((), ())),
                          preferred_element_type=jnp.float32)
    msg = jnp.maximum(acc + b_ref[...], 0.0)
    top = lax.bitcast_convert_type(
        msg[:HD, :].astype(jnp.bfloat16), jnp.uint16).astype(jnp.uint32)
    bot = lax.bitcast_convert_type(
        msg[HD:, :].astype(jnp.bfloat16), jnp.uint16).astype(jnp.uint32)
    o_ref[...] = (top | (bot << 16)).astype(jnp.int32)


def _edge_linear_packed(src_emb, W, b):
    return pl.pallas_call(
        _mmT_body,
        grid=(E // BM,),
        in_specs=[
            pl.BlockSpec((D, D), lambda i: (0, 0)),
            pl.BlockSpec((BM, D), lambda i: (i, 0)),
            pl.BlockSpec((D, 1), lambda i: (0, 0)),
        ],
        out_specs=pl.BlockSpec((HD, BM), lambda i: (0, i)),
        out_shape=jax.ShapeDtypeStruct((HD, E), jnp.int32),
    )(W, src_emb, b.reshape(D, 1))


NC = 2
NS = 16
RPT = HD // NS
EH = E // NC
CHUNK = 640
NCHUNKS = EH // CHUNK
NSTEP = CHUNK // 16


def _bmax(a, b):
    return plsc.bitcast(
        jnp.maximum(plsc.bitcast(a, jnp.bfloat16),
                    plsc.bitcast(b, jnp.bfloat16)), jnp.int32)


def _sc_body(blk_hbm, msgP_hbm, out_hbm, idx0, idx1, val0, val1, flag_v,
             sem_i0, sem_i1, sem_v0, sem_v1, *accs):
    sc = lax.axis_index("c")
    sid = lax.axis_index("s")
    row0 = sid * RPT
    e0 = sc * EH

    idxs = (idx0, idx1)
    vals = (val0, val1)
    sems_i = (sem_i0, sem_i1)
    sems_v = (sem_v0, sem_v1)

    zero16 = jnp.zeros((16,), jnp.int32)

    def zero_body(i, _):
        for c in range(RPT):
            accs[c][pl.ds(i * 16, 16)] = zero16
        return 0

    lax.fori_loop(0, PAD_N // 16, zero_body, 0)

    def issue(k, b):
        pltpu.async_copy(blk_hbm.at[pl.ds(e0 + k * CHUNK, CHUNK)],
                         idxs[b], sems_i[b])
        pltpu.async_copy(
            msgP_hbm.at[pl.ds(row0, RPT), pl.ds(e0 + k * CHUNK, CHUNK)],
            vals[b], sems_v[b])

    def wait(b):
        pltpu.make_async_copy(blk_hbm.at[pl.ds(e0, CHUNK)],
                              idxs[b], sems_i[b]).wait()
        pltpu.make_async_copy(
            msgP_hbm.at[pl.ds(row0, RPT), pl.ds(e0, CHUNK)],
            vals[b], sems_v[b]).wait()

    def process(k, b):
        idx_v = idxs[b]
        val_v = vals[b]

        def flag_body(s, _):
            d16 = idx_v[pl.ds(s * 16, 16)]
            _, last = plsc.scan_count(d16)
            flag_v[pl.ds(s * 16, 16)] = plsc.all_reduce_population_count(last)
            return 0

        lax.fori_loop(0, NSTEP, flag_body, 0)

        def step(j, _):
            d16 = idx_v[pl.ds(j * 16, 16)]
            nodup = flag_v[pl.ds(j * 16, 16)][0] == 16

            @pl.when(nodup)
            def _fast():
                for c in range(RPT):
                    v = val_v[c, pl.ds(j * 16, 16)]
                    a = plsc.load_gather(accs[c], [d16])
                    plsc.store_scatter(accs[c], [d16], _bmax(a, v))

            @pl.when(jnp.logical_not(nodup))
            def _slow():
                for c in range(RPT):
                    v = val_v[c, pl.ds(j * 16, 16)]

                    def cond(carry):
                        return jnp.any(carry[0])

                    def body(carry):
                        pend, m = carry
                        a = plsc.load_gather(accs[c], [d16])
                        m2 = _bmax(a, m)
                        plsc.store_scatter(accs[c], [d16], m2, mask=pend)
                        g2 = plsc.load_gather(accs[c], [d16])
                        return jnp.logical_and(pend, g2 != m2), m2

                    lax.while_loop(cond, body,
                                   (jnp.ones((16,), jnp.bool_), v))

            return 0

        lax.fori_loop(0, NSTEP, step, 0)

    issue(0, 0)

    def pair_body(t, _):
        k0 = t * 2

        wait(0)
        pl.when(k0 + 1 < NCHUNKS)(lambda: issue(k0 + 1, 1))
        process(k0, 0)

        @pl.when(k0 + 1 < NCHUNKS)
        def _odd():
            wait(1)
            pl.when(k0 + 2 < NCHUNKS)(lambda: issue(k0 + 2, 0))
            process(k0 + 1, 1)

        return 0

    lax.fori_loop(0, (NCHUNKS + 1) // 2, pair_body, 0)

    for c in range(RPT):
        pltpu.sync_copy(accs[c], out_hbm.at[sc, row0 + c])


@functools.partial(
    pl.kernel,
    out_type=jax.ShapeDtypeStruct((NC, HD, PAD_N), jnp.int32),
    mesh=plsc.VectorSubcoreMesh(core_axis_name="c", subcore_axis_name="s"),
    scratch_types=[
        pltpu.VMEM((CHUNK,), jnp.int32),
        pltpu.VMEM((CHUNK,), jnp.int32),
        pltpu.VMEM((RPT, CHUNK), jnp.int32),
        pltpu.VMEM((RPT, CHUNK), jnp.int32),
        pltpu.VMEM((CHUNK,), jnp.int32),
        pltpu.SemaphoreType.DMA,
        pltpu.SemaphoreType.DMA,
        pltpu.SemaphoreType.DMA,
        pltpu.SemaphoreType.DMA,
    ] + [pltpu.VMEM((PAD_N,), jnp.int32) for _ in range(RPT)],
    compiler_params=pltpu.CompilerParams(needs_layout_passes=False),
)
def _sc_scatter_max(blk_hbm, msgP_hbm, out_hbm, *scratch):
    _sc_body(blk_hbm, msgP_hbm, out_hbm, *scratch)


def _unpack(p):
    lo = lax.bitcast_convert_type(
        (p & 0xFFFF).astype(jnp.uint16), jnp.bfloat16)
    hi = lax.bitcast_convert_type(
        ((p >> 16) & 0xFFFF).astype(jnp.uint16), jnp.bfloat16)
    return lo, hi


def kernel(block, src_emb, src_emb_in, W, b):
    msgP = _edge_linear_packed(src_emb, W, b)
    parts = _sc_scatter_max(block.astype(jnp.int32), msgP)
    lo0, hi0 = _unpack(parts[0])
    lo1, hi1 = _unpack(parts[1])
    top = jnp.maximum(lo0, lo1).astype(jnp.float32)
    bot = jnp.maximum(hi0, hi1).astype(jnp.float32)
    hT = jnp.concatenate([top, bot], axis=0)
    return hT[:, :N_DST].T + src_emb[E:, :]


# speedup vs baseline: 1.0955x; 1.0955x over previous
import functools

import jax
import jax.numpy as jnp
from jax import lax
from jax.experimental import pallas as pl
from jax.experimental.pallas import tpu as pltpu
from jax.experimental.pallas import tpu_sc as plsc

E = 160000
D = 256
HD = D // 2
N_DST = 10000
PAD_N = 10240


BM = 1280


def _mmT_body(w_ref, x_ref, b_ref, o_ref):
    acc = lax.dot_general(w_ref[...].astype(jnp.bfloat16),
                          x_ref[...].astype(jnp.bfloat16),
                          (((1,), (1,)), ((), ())),
                          preferred_element_type=jnp.float32)
    msg = jnp.maximum(acc + b_ref[...], 0.0)
    top = lax.bitcast_convert_type(
        msg[:HD, :].astype(jnp.bfloat16), jnp.uint16).astype(jnp.uint32)
    bot = lax.bitcast_convert_type(
        msg[HD:, :].astype(jnp.bfloat16), jnp.uint16).astype(jnp.uint32)
    o_ref[...] = (top | (bot << 16)).astype(jnp.int32)


def _edge_linear_packed(src_emb, W, b, e_base, e_count):
    base_blk = e_base // BM
    return pl.pallas_call(
        _mmT_body,
        grid=(e_count // BM,),
        in_specs=[
            pl.BlockSpec((D, D), lambda i: (0, 0)),
            pl.BlockSpec((BM, D), lambda i: (base_blk + i, 0)),
            pl.BlockSpec((D, 1), lambda i: (0, 0)),
        ],
        out_specs=pl.BlockSpec((HD, BM), lambda i: (0, i)),
        out_shape=jax.ShapeDtypeStruct((HD, e_count), jnp.int32),
    )(W, src_emb, b.reshape(D, 1))


NC = 2
NS = 16
RPT = HD // NS
CHUNK = 640
NSTEP = CHUNK // 16
EA = 81920
EB = E - EA


def _bmax(a, b):
    return plsc.bitcast(
        jnp.maximum(plsc.bitcast(a, jnp.bfloat16),
                    plsc.bitcast(b, jnp.bfloat16)), jnp.int32)


def _make_sc_scatter(e_count, with_init):
    ehalf = e_count // 2
    nchunks = ehalf // CHUNK

    def body(blk_hbm, msgP_hbm, *rest):
        if with_init:
            init_hbm, out_hbm = rest[0], rest[1]
            scratch = rest[2:]
        else:
            out_hbm = rest[0]
            scratch = rest[1:]
        (idx0, idx1, val0, val1, flag_v,
         sem_i0, sem_i1, sem_v0, sem_v1, *accs) = scratch

        sc = lax.axis_index("c")
        sid = lax.axis_index("s")
        row0 = sid * RPT
        e0 = sc * ehalf

        idxs = (idx0, idx1)
        vals = (val0, val1)
        sems_i = (sem_i0, sem_i1)
        sems_v = (sem_v0, sem_v1)

        if with_init:
            for c in range(RPT):
                pltpu.sync_copy(init_hbm.at[sc, row0 + c], accs[c])
        else:
            zero16 = jnp.zeros((16,), jnp.int32)

            def zero_body(i, _):
                for c in range(RPT):
                    accs[c][pl.ds(i * 16, 16)] = zero16
                return 0

            lax.fori_loop(0, PAD_N // 16, zero_body, 0)

        def issue(k, b):
            pltpu.async_copy(blk_hbm.at[pl.ds(e0 + k * CHUNK, CHUNK)],
                             idxs[b], sems_i[b])
            pltpu.async_copy(
                msgP_hbm.at[pl.ds(row0, RPT), pl.ds(e0 + k * CHUNK, CHUNK)],
                vals[b], sems_v[b])

        def wait(b):
            pltpu.make_async_copy(blk_hbm.at[pl.ds(e0, CHUNK)],
                                  idxs[b], sems_i[b]).wait()
            pltpu.make_async_copy(
                msgP_hbm.at[pl.ds(row0, RPT), pl.ds(e0, CHUNK)],
                vals[b], sems_v[b]).wait()

        def process(k, b):
            idx_v = idxs[b]
            val_v = vals[b]

            def flag_body(s_, _):
                d16 = idx_v[pl.ds(s_ * 16, 16)]
                _, last = plsc.scan_count(d16)
                flag_v[pl.ds(s_ * 16, 16)] = (
                    plsc.all_reduce_population_count(last))
                return 0

            lax.fori_loop(0, NSTEP, flag_body, 0)

            def step(j, _):
                d16 = idx_v[pl.ds(j * 16, 16)]
                nodup = flag_v[pl.ds(j * 16, 16)][0] == 16

                @pl.when(nodup)
                def _fast():
                    for c in range(RPT):
                        v = val_v[c, pl.ds(j * 16, 16)]
                        a = plsc.load_gather(accs[c], [d16])
                        plsc.store_scatter(accs[c], [d16], _bmax(a, v))

                @pl.when(jnp.logical_not(nodup))
                def _slow():
                    for c in range(RPT):
                        v = val_v[c, pl.ds(j * 16, 16)]

                        def cond(carry):
                            return jnp.any(carry[0])

                        def body_w(carry):
                            pend, m = carry
                            a = plsc.load_gather(accs[c], [d16])
                            m2 = _bmax(a, m)
                            plsc.store_scatter(accs[c], [d16], m2, mask=pend)
                            g2 = plsc.load_gather(accs[c], [d16])
                            return jnp.logical_and(pend, g2 != m2), m2

                        lax.while_loop(cond, body_w,
                                       (jnp.ones((16,), jnp.bool_), v))

                return 0

            lax.fori_loop(0, NSTEP, step, 0)

        issue(0, 0)

        def pair_body(t, _):
            k0 = t * 2

            wait(0)
            pl.when(k0 + 1 < nchunks)(lambda: issue(k0 + 1, 1))
            process(k0, 0)

            @pl.when(k0 + 1 < nchunks)
            def _odd():
                wait(1)
                pl.when(k0 + 2 < nchunks)(lambda: issue(k0 + 2, 0))
                process(k0 + 1, 1)

            return 0

        lax.fori_loop(0, (nchunks + 1) // 2, pair_body, 0)

        for c in range(RPT):
            pltpu.sync_copy(accs[c], out_hbm.at[sc, row0 + c])

    return functools.partial(
        pl.kernel,
        out_type=jax.ShapeDtypeStruct((NC, HD, PAD_N), jnp.int32),
        mesh=plsc.VectorSubcoreMesh(core_axis_name="c",
                                    subcore_axis_name="s"),
        scratch_types=[
            pltpu.VMEM((CHUNK,), jnp.int32),
            pltpu.VMEM((CHUNK,), jnp.int32),
            pltpu.VMEM((RPT, CHUNK), jnp.int32),
            pltpu.VMEM((RPT, CHUNK), jnp.int32),
            pltpu.VMEM((CHUNK,), jnp.int32),
            pltpu.SemaphoreType.DMA,
            pltpu.SemaphoreType.DMA,
            pltpu.SemaphoreType.DMA,
            pltpu.SemaphoreType.DMA,
        ] + [pltpu.VMEM((PAD_N,), jnp.int32) for _ in range(RPT)],
        compiler_params=pltpu.CompilerParams(needs_layout_passes=False),
    )(body)


_sc_scatter_a = _make_sc_scatter(EA, with_init=False)
_sc_scatter_b = _make_sc_scatter(EB, with_init=True)


def _unpack(p):
    lo = lax.bitcast_convert_type(
        (p & 0xFFFF).astype(jnp.uint16), jnp.bfloat16)
    hi = lax.bitcast_convert_type(
        ((p >> 16) & 0xFFFF).astype(jnp.uint16), jnp.bfloat16)
    return lo, hi


def kernel(block, src_emb, src_emb_in, W, b):
    blk = block.astype(jnp.int32)
    msgP_a = _edge_linear_packed(src_emb, W, b, 0, EA)
    msgP_b = _edge_linear_packed(src_emb, W, b, EA, EB)
    p_a = _sc_scatter_a(blk[:EA], msgP_a)
    parts = _sc_scatter_b(blk[EA:], msgP_b, p_a)
    lo0, hi0 = _unpack(parts[0])
    lo1, hi1 = _unpack(parts[1])
    top = jnp.maximum(lo0, lo1).astype(jnp.float32)
    bot = jnp.maximum(hi0, hi1).astype(jnp.float32)
    hT = jnp.concatenate([top, bot], axis=0)
    return hT[:, :N_DST].T + src_emb[E:, :]


# speedup vs baseline: 1.1002x; 1.0043x over previous
import functools

import jax
import jax.numpy as jnp
from jax import lax
from jax.experimental import pallas as pl
from jax.experimental.pallas import tpu as pltpu
from jax.experimental.pallas import tpu_sc as plsc

E = 160000
D = 256
HD = D // 2
N_DST = 10000
PAD_N = 10240


BM = 1280


def _mmT_body(w_ref, x_ref, b_ref, o_ref):
    acc = lax.dot_general(w_ref[...].astype(jnp.bfloat16),
                          x_ref[...].astype(jnp.bfloat16),
                          (((1,), (1,)), ((), ())),
                          preferred_element_type=jnp.float32)
    msg = jnp.maximum(acc + b_ref[...], 0.0)
    top = lax.bitcast_convert_type(
        msg[:HD, :].astype(jnp.bfloat16), jnp.uint16).astype(jnp.uint32)
    bot = lax.bitcast_convert_type(
        msg[HD:, :].astype(jnp.bfloat16), jnp.uint16).astype(jnp.uint32)
    o_ref[...] = (top | (bot << 16)).astype(jnp.int32)


def _edge_linear_packed(src_emb, W, b, e_base, e_count):
    base_blk = e_base // BM
    return pl.pallas_call(
        _mmT_body,
        grid=(e_count // BM,),
        in_specs=[
            pl.BlockSpec((D, D), lambda i: (0, 0)),
            pl.BlockSpec((BM, D), lambda i: (base_blk + i, 0)),
            pl.BlockSpec((D, 1), lambda i: (0, 0)),
        ],
        out_specs=pl.BlockSpec((HD, BM), lambda i: (0, i)),
        out_shape=jax.ShapeDtypeStruct((HD, e_count), jnp.int32),
    )(W, src_emb, b.reshape(D, 1))


NC = 2
NS = 16
RPT = HD // NS
CHUNK = 640
NSTEP = CHUNK // 16
EA = 81920
EB = E - EA


def _bmax(a, b):
    return plsc.bitcast(
        jnp.maximum(plsc.bitcast(a, jnp.bfloat16),
                    plsc.bitcast(b, jnp.bfloat16)), jnp.int32)


def _make_sc_scatter(e_count, with_init):
    ehalf = e_count // 2
    nchunks = ehalf // CHUNK

    def body(blk_hbm, msgP_hbm, *rest):
        if with_init:
            init_hbm, out_hbm = rest[0], rest[1]
            scratch = rest[2:]
        else:
            out_hbm = rest[0]
            scratch = rest[1:]
        (idx0, idx1, val0, val1, flag_v,
         sem_i0, sem_i1, sem_v0, sem_v1, *accs) = scratch

        sc = lax.axis_index("c")
        sid = lax.axis_index("s")
        row0 = sid * RPT
        e0 = sc * ehalf

        idxs = (idx0, idx1)
        vals = (val0, val1)
        sems_i = (sem_i0, sem_i1)
        sems_v = (sem_v0, sem_v1)

        if with_init:
            for c in range(RPT):
                pltpu.sync_copy(init_hbm.at[sc, row0 + c], accs[c])
        else:
            zero16 = jnp.zeros((16,), jnp.int32)

            def zero_body(i, _):
                for c in range(RPT):
                    accs[c][pl.ds(i * 16, 16)] = zero16
                return 0

            lax.fori_loop(0, PAD_N // 16, zero_body, 0)

        def issue(k, b):
            pltpu.async_copy(blk_hbm.at[pl.ds(e0 + k * CHUNK, CHUNK)],
                             idxs[b], sems_i[b])
            pltpu.async_copy(
                msgP_hbm.at[pl.ds(row0, RPT), pl.ds(e0 + k * CHUNK, CHUNK)],
                vals[b], sems_v[b])

        def wait(b):
            pltpu.make_async_copy(blk_hbm.at[pl.ds(e0, CHUNK)],
                                  idxs[b], sems_i[b]).wait()
            pltpu.make_async_copy(
                msgP_hbm.at[pl.ds(row0, RPT), pl.ds(e0, CHUNK)],
                vals[b], sems_v[b]).wait()

        def process(k, b):
            idx_v = idxs[b]
            val_v = vals[b]

            def flag_body(s_, _):
                d16 = idx_v[pl.ds(s_ * 16, 16)]
                _, last = plsc.scan_count(d16)
                flag_v[pl.ds(s_ * 16, 16)] = (
                    plsc.all_reduce_population_count(last))
                return 0

            lax.fori_loop(0, NSTEP, flag_body, 0)

            def do_group(j):
                d16 = idx_v[pl.ds(j * 16, 16)]
                nodup = flag_v[pl.ds(j * 16, 16)][0] == 16

                def _fast():
                    for c in range(RPT):
                        v = val_v[c, pl.ds(j * 16, 16)]
                        a = plsc.load_gather(accs[c], [d16])
                        plsc.store_scatter(accs[c], [d16], _bmax(a, v))

                def _slow():
                    for c in range(RPT):
                        v = val_v[c, pl.ds(j * 16, 16)]

                        def cond(carry):
                            return jnp.any(carry[0])

                        def body_w(carry):
                            pend, m = carry
                            a = plsc.load_gather(accs[c], [d16])
                            m2 = _bmax(a, m)
                            plsc.store_scatter(accs[c], [d16], m2, mask=pend)
                            g2 = plsc.load_gather(accs[c], [d16])
                            return jnp.logical_and(pend, g2 != m2), m2

                        lax.while_loop(cond, body_w,
                                       (jnp.ones((16,), jnp.bool_), v))

                lax.cond(nodup, _fast, _slow)

            def step(j, _):
                do_group(j * 2)
                do_group(j * 2 + 1)
                return 0

            lax.fori_loop(0, NSTEP // 2, step, 0)

        issue(0, 0)

        def pair_body(t, _):
            k0 = t * 2

            wait(0)
            pl.when(k0 + 1 < nchunks)(lambda: issue(k0 + 1, 1))
            process(k0, 0)

            @pl.when(k0 + 1 < nchunks)
            def _odd():
                wait(1)
                pl.when(k0 + 2 < nchunks)(lambda: issue(k0 + 2, 0))
                process(k0 + 1, 1)

            return 0

        lax.fori_loop(0, (nchunks + 1) // 2, pair_body, 0)

        for c in range(RPT):
            pltpu.sync_copy(accs[c], out_hbm.at[sc, row0 + c])

    return functools.partial(
        pl.kernel,
        out_type=jax.ShapeDtypeStruct((NC, HD, PAD_N), jnp.int32),
        mesh=plsc.VectorSubcoreMesh(core_axis_name="c",
                                    subcore_axis_name="s"),
        scratch_types=[
            pltpu.VMEM((CHUNK,), jnp.int32),
            pltpu.VMEM((CHUNK,), jnp.int32),
            pltpu.VMEM((RPT, CHUNK), jnp.int32),
            pltpu.VMEM((RPT, CHUNK), jnp.int32),
            pltpu.VMEM((CHUNK,), jnp.int32),
            pltpu.SemaphoreType.DMA,
            pltpu.SemaphoreType.DMA,
            pltpu.SemaphoreType.DMA,
            pltpu.SemaphoreType.DMA,
        ] + [pltpu.VMEM((PAD_N,), jnp.int32) for _ in range(RPT)],
        compiler_params=pltpu.CompilerParams(needs_layout_passes=False),
    )(body)


_sc_scatter_a = _make_sc_scatter(EA, with_init=False)
_sc_scatter_b = _make_sc_scatter(EB, with_init=True)


def _unpack(p):
    lo = lax.bitcast_convert_type(
        (p & 0xFFFF).astype(jnp.uint16), jnp.bfloat16)
    hi = lax.bitcast_convert_type(
        ((p >> 16) & 0xFFFF).astype(jnp.uint16), jnp.bfloat16)
    return lo, hi


def kernel(block, src_emb, src_emb_in, W, b):
    blk = block.astype(jnp.int32)
    msgP_a = _edge_linear_packed(src_emb, W, b, 0, EA)
    msgP_b = _edge_linear_packed(src_emb, W, b, EA, EB)
    p_a = _sc_scatter_a(blk[:EA], msgP_a)
    parts = _sc_scatter_b(blk[EA:], msgP_b, p_a)
    lo0, hi0 = _unpack(parts[0])
    lo1, hi1 = _unpack(parts[1])
    top = jnp.maximum(lo0, lo1).astype(jnp.float32)
    bot = jnp.maximum(hi0, hi1).astype(jnp.float32)
    hT = jnp.concatenate([top, bot], axis=0)
    return hT[:, :N_DST].T + src_emb[E:, :]
